# baseline (speedup 1.0000x reference)
import functools

import jax
import jax.numpy as jnp
from jax import lax
from jax.experimental import pallas as pl
from jax.experimental.pallas import tpu as pltpu
from jax.experimental.pallas import tpu_sc as plsc

N = 10000
E = 160000
NW = 32
CH = 128
KCH = 40
EP = NW * KCH * CH
NACC = 10016
ZR = NACC // 16
RS = N // 16
TE = 2048
GE = EP // TE
TN = 2000
GN = N // TN
F32 = jnp.float32
BF16 = jnp.bfloat16
EPS = 1e-5


def _ln_tanh(h, g, be):
    mu = jnp.mean(h, axis=-1, keepdims=True)
    d = h - mu
    var = jnp.mean(d * d, axis=-1, keepdims=True)
    return jnp.tanh(d * lax.rsqrt(var + EPS) * g + be)


def _mm(a, w):
    return jnp.dot(a.astype(BF16), w.astype(BF16), preferred_element_type=F32)


def _tc_call(body, n_grid, data, dspecs, wts, out_shapes, ospecs):
    names = sorted(wts)
    warrs = [wts[n] for n in names]
    nd = len(data)
    nw = len(names)

    def wrapped(*refs):
        drefs = refs[:nd]
        wrefs = {n: r[...] for n, r in zip(names, refs[nd:nd + nw])}
        orefs = refs[nd + nw:]
        body(drefs, wrefs, orefs)

    wspecs = [
        pl.BlockSpec(a.shape, functools.partial(lambda i, nd_: (0,) * nd_, nd_=a.ndim))
        for a in warrs
    ]
    return pl.pallas_call(
        wrapped,
        grid=(n_grid,),
        in_specs=[*dspecs, *wspecs],
        out_specs=ospecs,
        out_shape=out_shapes,
    )(*data, *warrs)


def _espec(d):
    return pl.BlockSpec((TE, d), lambda i: (i, 0))


def _nspec(d):
    return pl.BlockSpec((TN, d), lambda i: (i, 0))


def _mlp_weights(p, prefix, splits):
    d = {}
    W1 = p["Ws"][0]
    for nm, a, b in splits:
        d[prefix + "1" + nm] = W1[a:b]
    d[prefix + "b1"] = p["bs"][0][None, :]
    d[prefix + "g1"] = p["g"][0][None, :]
    d[prefix + "e1"] = p["be"][0][None, :]
    d[prefix + "W2"] = p["Ws"][1]
    d[prefix + "b2"] = p["bs"][1][None, :]
    d[prefix + "g2"] = p["g"][1][None, :]
    d[prefix + "e2"] = p["be"][1][None, :]
    d[prefix + "W3"] = p["Ws"][2]
    d[prefix + "b3"] = p["bs"][2][None, :]
    return d


def _tail23(W, h1, prefix):
    h2 = _ln_tanh(_mm(h1, W[prefix + "W2"]) + W[prefix + "b2"],
                  W[prefix + "g2"], W[prefix + "e2"])
    return _mm(h2, W[prefix + "W3"]) + W[prefix + "b3"]


def _pad128(v):
    return jnp.concatenate([v, jnp.zeros_like(v)], axis=-1)


def _node_enc_body(d, W, o):
    h = _ln_tanh(_mm(d[0][...], W["ne1a"]) + W["neb1"], W["neg1"], W["nee1"])
    o[0][...] = _pad128(_tail23(W, h, "ne"))


def _pack_state(eh, el, w):
    return jnp.concatenate(
        [eh, el, w, jnp.zeros((eh.shape[0], 62), F32)], axis=-1)


def _init_edge_body(d, W, o):
    ixr, ixc, ea, xr, xc = (r[...] for r in d)
    xr = xr[:, :64].astype(F32)
    xc = xc[:, :64].astype(F32)
    h = _mm(ixr, W["ee1a"]) + _mm(ixc, W["ee1b"]) + _mm(ea, W["ee1c"]) + W["eeb1"]
    h = _ln_tanh(h, W["eeg1"], W["eee1"])
    eh = _tail23(W, h, "ee")
    h = (_mm(xr, W["ic1a"]) + _mm(xc, W["ic1b"]) + _mm(eh, W["ic1c"])
         + _mm(ixr, W["ic1d"]) + _mm(ixc, W["ic1e"]) + _mm(ea, W["ic1f"])
         + W["icb1"])
    h = _ln_tanh(h, W["icg1"], W["ice1"])
    el = jax.nn.sigmoid(_tail23(W, h, "ic"))
    c1 = _mm(ixr, W["em1d"]) + _mm(ixc, W["em1e"]) + _mm(ea, W["em1f"]) + W["emb1"]
    c2 = _mm(ixr, W["ec1d"]) + _mm(ixc, W["ec1e"]) + _mm(ea, W["ec1f"]) + W["ecb1"]
    h = el * W["gw1r"] + _mm(eh, W["gw1a"]) + W["gwb1"]
    h = _ln_tanh(h, W["gwg1"], W["gwe1"])
    w = _tail23(W, h, "gw")
    o[0][...] = _pack_state(eh, el, w)
    o[1][...] = c1.astype(BF16)
    o[2][...] = c2.astype(BF16)
    o[3][...] = jnp.concatenate([w * xr, w * eh], axis=-1)


def _sum_body(d, W, o):
    p = d[0][...]
    o[0][...] = p[0] + p[1]


def _m2_body(d, W, o):
    ohr, xc, st = (r[...] for r in d)
    eh = st[:, :64]
    w = st[:, 65:66]
    o[0][...] = w * ohr - (w * w) * jnp.concatenate([xc[:, :64], eh], axis=-1)


def _gat_x_body(d, W, o):
    p2 = d[0][...]
    oh, xe, x0 = (r[...] for r in d[1:])
    th = p2[0] + p2[1]
    h = (_mm(xe[:, :64], W["gx1a"]) + _mm(x0, W["gx1b"]) + _mm(oh, W["gx1c"])
         + _mm(th, W["gx1d"]) + W["gxb1"])
    h = _ln_tanh(h, W["gxg1"], W["gxe1"])
    o[0][...] = _pad128(_tail23(W, h, "gx"))


def _edge_fused_body(d, W, o):
    xr, xc, st, c1, c2 = (r[...] for r in d)
    xr = xr[:, :64].astype(F32)
    xc = xc[:, :64].astype(F32)
    eh = st[:, :64]
    el = st[:, 64:65]
    h = (_mm(xr, W["em1a"]) + _mm(xc, W["em1b"]) + _mm(eh, W["em1c"])
         + el * W["em1r"] + c1.astype(F32))
    h = _ln_tanh(h, W["emg1"], W["eme1"])
    eh_n = _tail23(W, h, "em")
    h = (_mm(xr, W["ec1a"]) + _mm(xc, W["ec1b"]) + _mm(eh_n, W["ec1c"])
         + el * W["ec1r"] + c2.astype(F32))
    h = _ln_tanh(h, W["ecg1"], W["ece1"])
    el_n = jax.nn.sigmoid(_tail23(W, h, "ec"))
    h = el_n * W["gw1r"] + _mm(eh_n, W["gw1a"]) + W["gwb1"]
    h = _ln_tanh(h, W["gwg1"], W["gwe1"])
    w = _tail23(W, h, "gw")
    o[0][...] = _pack_state(eh_n, el_n, w)
    o[1][...] = jnp.concatenate([w * xr, w * eh_n], axis=-1)


NB = 2
NWAVE = KCH // NB


@functools.lru_cache(maxsize=None)
def _gather_kernel(n_idx, D, dt):
    mesh = plsc.VectorSubcoreMesh(core_axis_name="c", subcore_axis_name="s")

    @functools.partial(
        pl.kernel,
        out_type=[jax.ShapeDtypeStruct((EP, D), dt) for _ in range(n_idx)],
        mesh=mesh,
        scratch_types=[
            pltpu.VMEM((KCH, CH), jnp.int32),
            *[pltpu.VMEM((CH, D), dt) for _ in range(NB)],
            *[pltpu.SemaphoreType.DMA for _ in range(NB)],
            *[pltpu.SemaphoreType.DMA for _ in range(NB)],
            pltpu.VMEM_SHARED((NACC, D), dt),
        ],
    )
    def gk(*refs):
        table_h = refs[0]
        idx_h = refs[1:1 + n_idx]
        out_h = refs[1 + n_idx:1 + 2 * n_idx]
        scr = refs[1 + 2 * n_idx:]
        idx_v = scr[0]
        bufs = scr[1:1 + NB]
        sin = scr[1 + NB:1 + 2 * NB]
        sout = scr[1 + 2 * NB:1 + 3 * NB]
        tab = scr[1 + 3 * NB]
        s = lax.axis_index("s")
        wid = s * 2 + lax.axis_index("c")
        base = wid * KCH

        pltpu.sync_copy(table_h.at[pl.ds(s * 624, 624)], tab.at[pl.ds(s * 624, 624)])

        @pl.when(s == 15)
        def _rem():
            pltpu.sync_copy(table_h.at[pl.ds(9984, 16)], tab.at[pl.ds(9984, 16)])

        plsc.subcore_barrier()

        for t in range(n_idx):
            pltpu.sync_copy(idx_h[t].at[wid], idx_v)
            for b in range(NB):
                pltpu.async_copy(tab.at[idx_v.at[b]], bufs[b], sin[b])

            def wave(wv, carry, t=t):
                for b in range(NB):
                    j = wv * NB + b
                    pltpu.make_async_copy(tab.at[idx_v.at[j]], bufs[b], sin[b]).wait()
                    pltpu.async_copy(bufs[b], out_h[t].at[pl.ds((base + j) * CH, CH)], sout[b])
                for b in range(NB):
                    j = wv * NB + b
                    pltpu.make_async_copy(bufs[b], out_h[t].at[pl.ds((base + j) * CH, CH)], sout[b]).wait()

                    @pl.when(wv + 1 < NWAVE)
                    def _next(b=b, wv=wv):
                        jn = (wv + 1) * NB + b
                        pltpu.async_copy(tab.at[idx_v.at[jn]], bufs[b], sin[b])

                return carry

            lax.fori_loop(0, NWAVE, wave, 0)

    return gk


@functools.lru_cache(maxsize=None)
def _scatter_kernel():
    NBS = 2
    NWS = KCH // NBS
    mesh = plsc.VectorSubcoreMesh(core_axis_name="c", subcore_axis_name="s")

    @functools.partial(
        pl.kernel,
        out_type=jax.ShapeDtypeStruct((2, N, 128), F32),
        mesh=mesh,
        scratch_types=[
            pltpu.VMEM((KCH, CH), jnp.int32),
            *[pltpu.VMEM((CH, 128), F32) for _ in range(NBS)],
            *[pltpu.SemaphoreType.DMA for _ in range(NBS)],
            pltpu.VMEM_SHARED((NACC, 128), F32),
        ],
    )
    def sk(m_h, col_h, z_h, out_h, *scr):
        idx_v = scr[0]
        bufs = scr[1:1 + NBS]
        sin = scr[1 + NBS:1 + 2 * NBS]
        acc = scr[1 + 2 * NBS]
        c = lax.axis_index("c")
        s = lax.axis_index("s")
        wid = s * 2 + c
        base = wid * KCH
        pltpu.sync_copy(z_h, acc.at[pl.ds(s * ZR, ZR)])
        plsc.subcore_barrier()
        pltpu.sync_copy(col_h.at[wid], idx_v)
        for b in range(NBS):
            pltpu.async_copy(m_h.at[pl.ds((base + b) * CH, CH)], bufs[b], sin[b])

        def wave(wv, carry):
            for b in range(NBS):
                j = wv * NBS + b
                pltpu.make_async_copy(m_h.at[pl.ds((base + j) * CH, CH)], bufs[b], sin[b]).wait()
                pltpu.sync_copy(bufs[b], acc.at[idx_v.at[j]], add=True)

                @pl.when(wv + 1 < NWS)
                def _next(b=b, wv=wv):
                    jn = (wv + 1) * NBS + b
                    pltpu.async_copy(m_h.at[pl.ds((base + jn) * CH, CH)], bufs[b], sin[b])

            return carry

        lax.fori_loop(0, NWS, wave, 0)
        plsc.subcore_barrier()
        pltpu.sync_copy(acc.at[pl.ds(s * 624, 624)], out_h.at[c, pl.ds(s * 624, 624)])

        @pl.when(s == 15)
        def _rem():
            pltpu.sync_copy(acc.at[pl.ds(9984, 16)], out_h.at[c, pl.ds(9984, 16)])

    return sk


def _sc_gather(table, idx3s, D, dt=F32):
    out = _gather_kernel(len(idx3s), D, dt)(table, *idx3s)
    return out if isinstance(out, (tuple, list)) else (out,)


def _sc_scatter(m, col3, zeros):
    return _scatter_kernel()(m, col3, zeros)


def kernel(x, edge_index, edge_attr, params):
    x0 = x
    row = edge_index[0]
    col = edge_index[1]
    pad = EP - E
    row3 = jnp.concatenate([row, jnp.zeros((pad,), jnp.int32)]).reshape(NW, KCH, CH)
    col3 = jnp.concatenate([col, jnp.full((pad,), N, jnp.int32)]).reshape(NW, KCH, CH)
    eap = jnp.concatenate([edge_attr, jnp.zeros((pad, 16), F32)], axis=0)
    zeros = jnp.zeros((ZR, 128), F32)

    P = params
    w_ne = _mlp_weights(P["node_encoder"], "ne", [("a", 0, 128)])
    w_ee = _mlp_weights(P["edge_encoder"], "ee",
                        [("a", 0, 128), ("b", 128, 256), ("c", 256, 272)])
    w_ic = _mlp_weights(P["init_cls"], "ic",
                        [("a", 0, 64), ("b", 64, 128), ("c", 128, 192),
                         ("d", 192, 320), ("e", 320, 448), ("f", 448, 464)])
    w_gx = _mlp_weights(P["gat_x_mlp"], "gx",
                        [("a", 0, 64), ("b", 64, 192), ("c", 192, 320),
                         ("d", 320, 448)])
    w_gw = _mlp_weights(P["gat_w_mlp"], "gw", [("r", 0, 1), ("a", 1, 65)])
    w_em = _mlp_weights(P["edge_mlp"], "em",
                        [("a", 0, 64), ("b", 64, 128), ("c", 128, 192),
                         ("r", 192, 193), ("d", 193, 321), ("e", 321, 449),
                         ("f", 449, 465)])
    w_ec = _mlp_weights(P["edge_cls"], "ec",
                        [("a", 0, 64), ("b", 64, 128), ("c", 128, 192),
                         ("r", 192, 193), ("d", 193, 321), ("e", 321, 449),
                         ("f", 449, 465)])

    ixr, ixc = _sc_gather(x0, (row3, col3), 128)
    xe = _tc_call(_node_enc_body, GN, [x0], [_nspec(128)], w_ne,
                  [jax.ShapeDtypeStruct((N, 128), F32)], [_nspec(128)])[0]
    xr, xc = _sc_gather(xe, (row3, col3), 128)

    w_init = {}
    for dd in (w_ee, w_ic, w_gw):
        w_init.update(dd)
    for k in ("em1d", "em1e", "em1f", "emb1"):
        w_init[k] = w_em[k]
    for k in ("ec1d", "ec1e", "ec1f", "ecb1"):
        w_init[k] = w_ec[k]
    st, c1, c2, m1 = _tc_call(
        _init_edge_body, GE,
        [ixr, ixc, eap, xr, xc],
        [_espec(128), _espec(128), _espec(16), _espec(128), _espec(128)],
        w_init,
        [jax.ShapeDtypeStruct((EP, 128), F32),
         jax.ShapeDtypeStruct((EP, 128), BF16),
         jax.ShapeDtypeStruct((EP, 128), BF16),
         jax.ShapeDtypeStruct((EP, 128), F32)],
        [_espec(128), _espec(128), _espec(128), _espec(128)])

    w_fused = {}
    for dd in (w_em, w_ec, w_gw):
        w_fused.update(dd)

    p2spec = pl.BlockSpec((2, TN, 128), lambda i: (0, i, 0))

    for _ in range(6):
        parts = _sc_scatter(m1, col3, zeros)
        one_hop = _tc_call(_sum_body, GN, [parts], [p2spec], {},
                           [jax.ShapeDtypeStruct((N, 128), F32)], [_nspec(128)])[0]
        (ohr,) = _sc_gather(one_hop, (row3,), 128)
        m2 = _tc_call(_m2_body, GE, [ohr, xc, st],
                      [_espec(128), _espec(128), _espec(128)], {},
                      [jax.ShapeDtypeStruct((EP, 128), F32)], [_espec(128)])[0]
        parts2 = _sc_scatter(m2, col3, zeros)
        xe = _tc_call(_gat_x_body, GN, [parts2, one_hop, xe, x0],
                      [p2spec, _nspec(128), _nspec(128), _nspec(128)], w_gx,
                      [jax.ShapeDtypeStruct((N, 128), F32)], [_nspec(128)])[0]
        xr, xc = _sc_gather(xe, (row3, col3), 128)
        st, m1 = _tc_call(
            _edge_fused_body, GE,
            [xr, xc, st, c1, c2],
            [_espec(128), _espec(128), _espec(128), _espec(128), _espec(128)],
            w_fused,
            [jax.ShapeDtypeStruct((EP, 128), F32),
             jax.ShapeDtypeStruct((EP, 128), F32)],
            [_espec(128), _espec(128)])

    return st[:E, 64]


# speedup vs baseline: 1.0441x; 1.0441x over previous
import functools

import jax
import jax.numpy as jnp
from jax import lax
from jax.experimental import pallas as pl
from jax.experimental.pallas import tpu as pltpu
from jax.experimental.pallas import tpu_sc as plsc

N = 10000
E = 160000
NW = 32
CH = 128
KCH = 40
EP = NW * KCH * CH
NACC = 10016
ZR = NACC // 16
RS = N // 16
TE = 2048
GE = EP // TE
TN = 2000
GN = N // TN
F32 = jnp.float32
BF16 = jnp.bfloat16
EPS = 1e-5


def _ln_tanh(h, g, be):
    mu = jnp.mean(h, axis=-1, keepdims=True)
    d = h - mu
    var = jnp.mean(d * d, axis=-1, keepdims=True)
    return jnp.tanh(d * lax.rsqrt(var + EPS) * g + be)


def _mm(a, w):
    return jnp.dot(a.astype(BF16), w.astype(BF16), preferred_element_type=F32)


def _tc_call(body, n_grid, data, dspecs, wts, out_shapes, ospecs):
    names = sorted(wts)
    warrs = [wts[n] for n in names]
    nd = len(data)
    nw = len(names)

    def wrapped(*refs):
        drefs = refs[:nd]
        wrefs = {n: r[...] for n, r in zip(names, refs[nd:nd + nw])}
        orefs = refs[nd + nw:]
        body(drefs, wrefs, orefs)

    wspecs = [
        pl.BlockSpec(a.shape, functools.partial(lambda i, nd_: (0,) * nd_, nd_=a.ndim))
        for a in warrs
    ]
    return pl.pallas_call(
        wrapped,
        grid=(n_grid,),
        in_specs=[*dspecs, *wspecs],
        out_specs=ospecs,
        out_shape=out_shapes,
    )(*data, *warrs)


def _espec(d):
    return pl.BlockSpec((TE, d), lambda i: (i, 0))


def _nspec(d):
    return pl.BlockSpec((TN, d), lambda i: (i, 0))


def _mlp_weights(p, prefix, splits):
    d = {}
    W1 = p["Ws"][0]
    for nm, a, b in splits:
        d[prefix + "1" + nm] = W1[a:b]
    d[prefix + "b1"] = p["bs"][0][None, :]
    d[prefix + "g1"] = p["g"][0][None, :]
    d[prefix + "e1"] = p["be"][0][None, :]
    d[prefix + "W2"] = p["Ws"][1]
    d[prefix + "b2"] = p["bs"][1][None, :]
    d[prefix + "g2"] = p["g"][1][None, :]
    d[prefix + "e2"] = p["be"][1][None, :]
    d[prefix + "W3"] = p["Ws"][2]
    d[prefix + "b3"] = p["bs"][2][None, :]
    return d


def _tail23(W, h1, prefix):
    h2 = _ln_tanh(_mm(h1, W[prefix + "W2"]) + W[prefix + "b2"],
                  W[prefix + "g2"], W[prefix + "e2"])
    return _mm(h2, W[prefix + "W3"]) + W[prefix + "b3"]


def _pad128(v):
    return jnp.concatenate([v, jnp.zeros_like(v)], axis=-1)


def _node_enc_body(d, W, o):
    h = _ln_tanh(_mm(d[0][...], W["ne1a"]) + W["neb1"], W["neg1"], W["nee1"])
    o[0][...] = _pad128(_tail23(W, h, "ne"))


def _init_edge_body(d, W, o):
    ixr, ixc, ea, xr, xc = (r[...] for r in d)
    xr = xr[:, :64].astype(F32)
    xc = xc[:, :64].astype(F32)
    h = _mm(ixr, W["ee1a"]) + _mm(ixc, W["ee1b"]) + _mm(ea, W["ee1c"]) + W["eeb1"]
    h = _ln_tanh(h, W["eeg1"], W["eee1"])
    eh = _tail23(W, h, "ee")
    h = (_mm(xr, W["ic1a"]) + _mm(xc, W["ic1b"]) + _mm(eh, W["ic1c"])
         + _mm(ixr, W["ic1d"]) + _mm(ixc, W["ic1e"]) + _mm(ea, W["ic1f"])
         + W["icb1"])
    h = _ln_tanh(h, W["icg1"], W["ice1"])
    el = jax.nn.sigmoid(_tail23(W, h, "ic"))
    c1 = _mm(ixr, W["em1d"]) + _mm(ixc, W["em1e"]) + _mm(ea, W["em1f"]) + W["emb1"]
    c2 = _mm(ixr, W["ec1d"]) + _mm(ixc, W["ec1e"]) + _mm(ea, W["ec1f"]) + W["ecb1"]
    h = el * W["gw1r"] + _mm(eh, W["gw1a"]) + W["gwb1"]
    h = _ln_tanh(h, W["gwg1"], W["gwe1"])
    w = _tail23(W, h, "gw")
    o[0][...] = eh
    o[1][...] = el
    o[2][...] = c1.astype(BF16)
    o[3][...] = c2.astype(BF16)
    o[4][...] = w
    o[5][...] = jnp.concatenate([w * xr, w * eh], axis=-1)


def _sum_body(d, W, o):
    p = d[0][...]
    o[0][...] = p[0] + p[1]


def _m2_body(d, W, o):
    ohr, xc, eh, w = (r[...] for r in d)
    o[0][...] = w * ohr - (w * w) * jnp.concatenate([xc[:, :64], eh], axis=-1)


def _gat_x_body(d, W, o):
    p2 = d[0][...]
    oh, xe, x0 = (r[...] for r in d[1:])
    th = p2[0] + p2[1]
    h = (_mm(xe[:, :64], W["gx1a"]) + _mm(x0, W["gx1b"]) + _mm(oh, W["gx1c"])
         + _mm(th, W["gx1d"]) + W["gxb1"])
    h = _ln_tanh(h, W["gxg1"], W["gxe1"])
    o[0][...] = _pad128(_tail23(W, h, "gx"))


def _edge_fused_body(d, W, o):
    xr, xc, eh, el, c1, c2 = (r[...] for r in d)
    xr = xr[:, :64].astype(F32)
    xc = xc[:, :64].astype(F32)
    h = (_mm(xr, W["em1a"]) + _mm(xc, W["em1b"]) + _mm(eh, W["em1c"])
         + el * W["em1r"] + c1.astype(F32))
    h = _ln_tanh(h, W["emg1"], W["eme1"])
    eh_n = _tail23(W, h, "em")
    h = (_mm(xr, W["ec1a"]) + _mm(xc, W["ec1b"]) + _mm(eh_n, W["ec1c"])
         + el * W["ec1r"] + c2.astype(F32))
    h = _ln_tanh(h, W["ecg1"], W["ece1"])
    el_n = jax.nn.sigmoid(_tail23(W, h, "ec"))
    h = el_n * W["gw1r"] + _mm(eh_n, W["gw1a"]) + W["gwb1"]
    h = _ln_tanh(h, W["gwg1"], W["gwe1"])
    w = _tail23(W, h, "gw")
    o[0][...] = eh_n
    o[1][...] = el_n
    o[2][...] = w
    o[3][...] = jnp.concatenate([w * xr, w * eh_n], axis=-1)


NB = 2
NWAVE = KCH // NB


@functools.lru_cache(maxsize=None)
def _gather_kernel(n_idx, D, dt):
    mesh = plsc.VectorSubcoreMesh(core_axis_name="c", subcore_axis_name="s")

    @functools.partial(
        pl.kernel,
        out_type=[jax.ShapeDtypeStruct((EP, D), dt) for _ in range(n_idx)],
        mesh=mesh,
        scratch_types=[
            pltpu.VMEM((KCH, CH), jnp.int32),
            *[pltpu.VMEM((CH, D), dt) for _ in range(NB)],
            *[pltpu.SemaphoreType.DMA for _ in range(NB)],
            *[pltpu.SemaphoreType.DMA for _ in range(NB)],
            pltpu.VMEM_SHARED((NACC, D), dt),
        ],
    )
    def gk(*refs):
        table_h = refs[0]
        idx_h = refs[1:1 + n_idx]
        out_h = refs[1 + n_idx:1 + 2 * n_idx]
        scr = refs[1 + 2 * n_idx:]
        idx_v = scr[0]
        bufs = scr[1:1 + NB]
        sin = scr[1 + NB:1 + 2 * NB]
        sout = scr[1 + 2 * NB:1 + 3 * NB]
        tab = scr[1 + 3 * NB]
        s = lax.axis_index("s")
        wid = s * 2 + lax.axis_index("c")
        base = wid * KCH

        pltpu.sync_copy(table_h.at[pl.ds(s * 624, 624)], tab.at[pl.ds(s * 624, 624)])

        @pl.when(s == 15)
        def _rem():
            pltpu.sync_copy(table_h.at[pl.ds(9984, 16)], tab.at[pl.ds(9984, 16)])

        plsc.subcore_barrier()

        for t in range(n_idx):
            pltpu.sync_copy(idx_h[t].at[wid], idx_v)
            for b in range(NB):
                pltpu.async_copy(tab.at[idx_v.at[b]], bufs[b], sin[b])

            def wave(wv, carry, t=t):
                for b in range(NB):
                    j = wv * NB + b
                    pltpu.make_async_copy(tab.at[idx_v.at[j]], bufs[b], sin[b]).wait()
                    pltpu.async_copy(bufs[b], out_h[t].at[pl.ds((base + j) * CH, CH)], sout[b])
                for b in range(NB):
                    j = wv * NB + b
                    pltpu.make_async_copy(bufs[b], out_h[t].at[pl.ds((base + j) * CH, CH)], sout[b]).wait()

                    @pl.when(wv + 1 < NWAVE)
                    def _next(b=b, wv=wv):
                        jn = (wv + 1) * NB + b
                        pltpu.async_copy(tab.at[idx_v.at[jn]], bufs[b], sin[b])

                return carry

            lax.fori_loop(0, NWAVE, wave, 0)

    return gk


@functools.lru_cache(maxsize=None)
def _scatter_kernel():
    NBS = 2
    NWS = KCH // NBS
    mesh = plsc.VectorSubcoreMesh(core_axis_name="c", subcore_axis_name="s")

    @functools.partial(
        pl.kernel,
        out_type=jax.ShapeDtypeStruct((2, N, 128), F32),
        mesh=mesh,
        scratch_types=[
            pltpu.VMEM((KCH, CH), jnp.int32),
            *[pltpu.VMEM((CH, 128), F32) for _ in range(NBS)],
            *[pltpu.SemaphoreType.DMA for _ in range(NBS)],
            pltpu.VMEM_SHARED((NACC, 128), F32),
        ],
    )
    def sk(m_h, col_h, z_h, out_h, *scr):
        idx_v = scr[0]
        bufs = scr[1:1 + NBS]
        sin = scr[1 + NBS:1 + 2 * NBS]
        acc = scr[1 + 2 * NBS]
        c = lax.axis_index("c")
        s = lax.axis_index("s")
        wid = s * 2 + c
        base = wid * KCH
        pltpu.sync_copy(z_h, acc.at[pl.ds(s * ZR, ZR)])
        plsc.subcore_barrier()
        pltpu.sync_copy(col_h.at[wid], idx_v)
        for b in range(NBS):
            pltpu.async_copy(m_h.at[pl.ds((base + b) * CH, CH)], bufs[b], sin[b])

        def wave(wv, carry):
            for b in range(NBS):
                j = wv * NBS + b
                pltpu.make_async_copy(m_h.at[pl.ds((base + j) * CH, CH)], bufs[b], sin[b]).wait()
                pltpu.sync_copy(bufs[b], acc.at[idx_v.at[j]], add=True)

                @pl.when(wv + 1 < NWS)
                def _next(b=b, wv=wv):
                    jn = (wv + 1) * NBS + b
                    pltpu.async_copy(m_h.at[pl.ds((base + jn) * CH, CH)], bufs[b], sin[b])

            return carry

        lax.fori_loop(0, NWS, wave, 0)
        plsc.subcore_barrier()
        pltpu.sync_copy(acc.at[pl.ds(s * 624, 624)], out_h.at[c, pl.ds(s * 624, 624)])

        @pl.when(s == 15)
        def _rem():
            pltpu.sync_copy(acc.at[pl.ds(9984, 16)], out_h.at[c, pl.ds(9984, 16)])

    return sk


def _sc_gather(table, idx3s, D, dt=F32):
    out = _gather_kernel(len(idx3s), D, dt)(table, *idx3s)
    return out if isinstance(out, (tuple, list)) else (out,)


def _sc_scatter(m, col3, zeros):
    return _scatter_kernel()(m, col3, zeros)


def kernel(x, edge_index, edge_attr, params):
    x0 = x
    row = edge_index[0]
    col = edge_index[1]
    pad = EP - E
    row3 = jnp.concatenate([row, jnp.zeros((pad,), jnp.int32)]).reshape(NW, KCH, CH)
    col3 = jnp.concatenate([col, jnp.full((pad,), N, jnp.int32)]).reshape(NW, KCH, CH)
    eap = jnp.concatenate([edge_attr, jnp.zeros((pad, 16), F32)], axis=0)
    zeros = jnp.zeros((ZR, 128), F32)

    P = params
    w_ne = _mlp_weights(P["node_encoder"], "ne", [("a", 0, 128)])
    w_ee = _mlp_weights(P["edge_encoder"], "ee",
                        [("a", 0, 128), ("b", 128, 256), ("c", 256, 272)])
    w_ic = _mlp_weights(P["init_cls"], "ic",
                        [("a", 0, 64), ("b", 64, 128), ("c", 128, 192),
                         ("d", 192, 320), ("e", 320, 448), ("f", 448, 464)])
    w_gx = _mlp_weights(P["gat_x_mlp"], "gx",
                        [("a", 0, 64), ("b", 64, 192), ("c", 192, 320),
                         ("d", 320, 448)])
    w_gw = _mlp_weights(P["gat_w_mlp"], "gw", [("r", 0, 1), ("a", 1, 65)])
    w_em = _mlp_weights(P["edge_mlp"], "em",
                        [("a", 0, 64), ("b", 64, 128), ("c", 128, 192),
                         ("r", 192, 193), ("d", 193, 321), ("e", 321, 449),
                         ("f", 449, 465)])
    w_ec = _mlp_weights(P["edge_cls"], "ec",
                        [("a", 0, 64), ("b", 64, 128), ("c", 128, 192),
                         ("r", 192, 193), ("d", 193, 321), ("e", 321, 449),
                         ("f", 449, 465)])

    ixr, ixc = _sc_gather(x0, (row3, col3), 128)
    xe = _tc_call(_node_enc_body, GN, [x0], [_nspec(128)], w_ne,
                  [jax.ShapeDtypeStruct((N, 128), F32)], [_nspec(128)])[0]
    xr, xc = _sc_gather(xe, (row3, col3), 128)

    w_init = {}
    for dd in (w_ee, w_ic, w_gw):
        w_init.update(dd)
    for k in ("em1d", "em1e", "em1f", "emb1"):
        w_init[k] = w_em[k]
    for k in ("ec1d", "ec1e", "ec1f", "ecb1"):
        w_init[k] = w_ec[k]
    eh, el, c1, c2, w, m1 = _tc_call(
        _init_edge_body, GE,
        [ixr, ixc, eap, xr, xc],
        [_espec(128), _espec(128), _espec(16), _espec(128), _espec(128)],
        w_init,
        [jax.ShapeDtypeStruct((EP, 64), F32),
         jax.ShapeDtypeStruct((EP, 1), F32),
         jax.ShapeDtypeStruct((EP, 128), BF16),
         jax.ShapeDtypeStruct((EP, 128), BF16),
         jax.ShapeDtypeStruct((EP, 1), F32),
         jax.ShapeDtypeStruct((EP, 128), F32)],
        [_espec(64), _espec(1), _espec(128), _espec(128), _espec(1), _espec(128)])

    w_fused = {}
    for dd in (w_em, w_ec, w_gw):
        w_fused.update(dd)

    p2spec = pl.BlockSpec((2, TN, 128), lambda i: (0, i, 0))

    for _ in range(6):
        parts = _sc_scatter(m1, col3, zeros)
        one_hop = _tc_call(_sum_body, GN, [parts], [p2spec], {},
                           [jax.ShapeDtypeStruct((N, 128), F32)], [_nspec(128)])[0]
        (ohr,) = _sc_gather(one_hop, (row3,), 128)
        m2 = _tc_call(_m2_body, GE, [ohr, xc, eh, w],
                      [_espec(128), _espec(128), _espec(64), _espec(1)], {},
                      [jax.ShapeDtypeStruct((EP, 128), F32)], [_espec(128)])[0]
        parts2 = _sc_scatter(m2, col3, zeros)
        xe = _tc_call(_gat_x_body, GN, [parts2, one_hop, xe, x0],
                      [p2spec, _nspec(128), _nspec(128), _nspec(128)], w_gx,
                      [jax.ShapeDtypeStruct((N, 128), F32)], [_nspec(128)])[0]
        xr, xc = _sc_gather(xe, (row3, col3), 128)
        eh, el, w, m1 = _tc_call(
            _edge_fused_body, GE,
            [xr, xc, eh, el, c1, c2],
            [_espec(128), _espec(128), _espec(64), _espec(1), _espec(128), _espec(128)],
            w_fused,
            [jax.ShapeDtypeStruct((EP, 64), F32),
             jax.ShapeDtypeStruct((EP, 1), F32),
             jax.ShapeDtypeStruct((EP, 1), F32),
             jax.ShapeDtypeStruct((EP, 128), F32)],
            [_espec(64), _espec(1), _espec(1), _espec(128)])

    return el[:E, 0]


# speedup vs baseline: 1.0842x; 1.0384x over previous
import functools

import jax
import jax.numpy as jnp
from jax import lax
from jax.experimental import pallas as pl
from jax.experimental.pallas import tpu as pltpu
from jax.experimental.pallas import tpu_sc as plsc

N = 10000
E = 160000
NW = 32
CH = 128
KCH = 40
EP = NW * KCH * CH
NACC = 10016
ZR = NACC // 16
RS = N // 16
TE = 4096
GE = EP // TE
TN = 2000
GN = N // TN
F32 = jnp.float32
BF16 = jnp.bfloat16
EPS = 1e-5


def _ln_tanh(h, g, be):
    mu = jnp.mean(h, axis=-1, keepdims=True)
    d = h - mu
    var = jnp.mean(d * d, axis=-1, keepdims=True)
    return jnp.tanh(d * lax.rsqrt(var + EPS) * g + be)


def _mm(a, w):
    return jnp.dot(a.astype(BF16), w, preferred_element_type=F32)


def _tc_call(body, n_grid, data, dspecs, wts, out_shapes, ospecs):
    names = sorted(wts)
    warrs = [wts[n] for n in names]
    nd = len(data)
    nw = len(names)

    def wrapped(*refs):
        drefs = refs[:nd]
        wrefs = {n: r[...] for n, r in zip(names, refs[nd:nd + nw])}
        orefs = refs[nd + nw:]
        body(drefs, wrefs, orefs)

    wspecs = [
        pl.BlockSpec(a.shape, functools.partial(lambda i, nd_: (0,) * nd_, nd_=a.ndim))
        for a in warrs
    ]
    return pl.pallas_call(
        wrapped,
        grid=(n_grid,),
        in_specs=[*dspecs, *wspecs],
        out_specs=ospecs,
        out_shape=out_shapes,
    )(*data, *warrs)


def _espec(d):
    return pl.BlockSpec((TE, d), lambda i: (i, 0))


def _nspec(d):
    return pl.BlockSpec((TN, d), lambda i: (i, 0))


def _mlp_weights(p, prefix, splits):
    d = {}
    W1 = p["Ws"][0]
    for nm, a, b in splits:
        w = W1[a:b]
        d[prefix + "1" + nm] = w if nm == "r" else w.astype(BF16)
    d[prefix + "b1"] = p["bs"][0][None, :]
    d[prefix + "g1"] = p["g"][0][None, :]
    d[prefix + "e1"] = p["be"][0][None, :]
    d[prefix + "W2"] = p["Ws"][1].astype(BF16)
    d[prefix + "b2"] = p["bs"][1][None, :]
    d[prefix + "g2"] = p["g"][1][None, :]
    d[prefix + "e2"] = p["be"][1][None, :]
    d[prefix + "W3"] = p["Ws"][2].astype(BF16)
    d[prefix + "b3"] = p["bs"][2][None, :]
    return d


def _tail23(W, h1, prefix):
    h2 = _ln_tanh(_mm(h1, W[prefix + "W2"]) + W[prefix + "b2"],
                  W[prefix + "g2"], W[prefix + "e2"])
    return _mm(h2, W[prefix + "W3"]) + W[prefix + "b3"]


def _pad128(v):
    return jnp.concatenate([v, jnp.zeros_like(v)], axis=-1)


def _node_enc_body(d, W, o):
    h = _ln_tanh(_mm(d[0][...], W["ne1a"]) + W["neb1"], W["neg1"], W["nee1"])
    o[0][...] = _pad128(_tail23(W, h, "ne"))


def _init_edge_body(d, W, o):
    ixr, ixc, ea, xr, xc = (r[...] for r in d)
    xr = xr[:, :64].astype(F32)
    xc = xc[:, :64].astype(F32)
    h = _mm(ixr, W["ee1a"]) + _mm(ixc, W["ee1b"]) + _mm(ea, W["ee1c"]) + W["eeb1"]
    h = _ln_tanh(h, W["eeg1"], W["eee1"])
    eh = _tail23(W, h, "ee")
    h = (_mm(xr, W["ic1a"]) + _mm(xc, W["ic1b"]) + _mm(eh, W["ic1c"])
         + _mm(ixr, W["ic1d"]) + _mm(ixc, W["ic1e"]) + _mm(ea, W["ic1f"])
         + W["icb1"])
    h = _ln_tanh(h, W["icg1"], W["ice1"])
    el = jax.nn.sigmoid(_tail23(W, h, "ic"))
    c1 = _mm(ixr, W["em1d"]) + _mm(ixc, W["em1e"]) + _mm(ea, W["em1f"]) + W["emb1"]
    c2 = _mm(ixr, W["ec1d"]) + _mm(ixc, W["ec1e"]) + _mm(ea, W["ec1f"]) + W["ecb1"]
    h = el * W["gw1r"] + _mm(eh, W["gw1a"]) + W["gwb1"]
    h = _ln_tanh(h, W["gwg1"], W["gwe1"])
    w = _tail23(W, h, "gw")
    o[0][...] = eh
    o[1][...] = el
    o[2][...] = c1.astype(BF16)
    o[3][...] = c2.astype(BF16)
    o[4][...] = w
    o[5][...] = jnp.concatenate([w * xr, w * eh], axis=-1)


def _sum_body(d, W, o):
    p = d[0][...]
    o[0][...] = p[0] + p[1]


def _m2_body(d, W, o):
    ohr, xc, eh, w = (r[...] for r in d)
    o[0][...] = w * ohr - (w * w) * jnp.concatenate([xc[:, :64], eh], axis=-1)


def _gat_x_body(d, W, o):
    p2 = d[0][...]
    oh, xe, x0 = (r[...] for r in d[1:])
    th = p2[0] + p2[1]
    h = (_mm(xe[:, :64], W["gx1a"]) + _mm(x0, W["gx1b"]) + _mm(oh, W["gx1c"])
         + _mm(th, W["gx1d"]) + W["gxb1"])
    h = _ln_tanh(h, W["gxg1"], W["gxe1"])
    o[0][...] = _pad128(_tail23(W, h, "gx"))


def _edge_fused_body(d, W, o):
    xr, xc, eh, el, c1, c2 = (r[...] for r in d)
    xr = xr[:, :64].astype(F32)
    xc = xc[:, :64].astype(F32)
    h = (_mm(xr, W["em1a"]) + _mm(xc, W["em1b"]) + _mm(eh, W["em1c"])
         + el * W["em1r"] + c1.astype(F32))
    h = _ln_tanh(h, W["emg1"], W["eme1"])
    eh_n = _tail23(W, h, "em")
    h = (_mm(xr, W["ec1a"]) + _mm(xc, W["ec1b"]) + _mm(eh_n, W["ec1c"])
         + el * W["ec1r"] + c2.astype(F32))
    h = _ln_tanh(h, W["ecg1"], W["ece1"])
    el_n = jax.nn.sigmoid(_tail23(W, h, "ec"))
    h = el_n * W["gw1r"] + _mm(eh_n, W["gw1a"]) + W["gwb1"]
    h = _ln_tanh(h, W["gwg1"], W["gwe1"])
    w = _tail23(W, h, "gw")
    o[0][...] = eh_n
    o[1][...] = el_n
    o[2][...] = w
    o[3][...] = jnp.concatenate([w * xr, w * eh_n], axis=-1)


NB = 2
NWAVE = KCH // NB


@functools.lru_cache(maxsize=None)
def _gather_kernel(n_idx, D, dt):
    mesh = plsc.VectorSubcoreMesh(core_axis_name="c", subcore_axis_name="s")

    @functools.partial(
        pl.kernel,
        out_type=[jax.ShapeDtypeStruct((EP, D), dt) for _ in range(n_idx)],
        mesh=mesh,
        scratch_types=[
            pltpu.VMEM((KCH, CH), jnp.int32),
            *[pltpu.VMEM((CH, D), dt) for _ in range(NB)],
            *[pltpu.SemaphoreType.DMA for _ in range(NB)],
            *[pltpu.SemaphoreType.DMA for _ in range(NB)],
            pltpu.VMEM_SHARED((NACC, D), dt),
        ],
    )
    def gk(*refs):
        table_h = refs[0]
        idx_h = refs[1:1 + n_idx]
        out_h = refs[1 + n_idx:1 + 2 * n_idx]
        scr = refs[1 + 2 * n_idx:]
        idx_v = scr[0]
        bufs = scr[1:1 + NB]
        sin = scr[1 + NB:1 + 2 * NB]
        sout = scr[1 + 2 * NB:1 + 3 * NB]
        tab = scr[1 + 3 * NB]
        s = lax.axis_index("s")
        wid = s * 2 + lax.axis_index("c")
        base = wid * KCH

        pltpu.sync_copy(table_h.at[pl.ds(s * 624, 624)], tab.at[pl.ds(s * 624, 624)])

        @pl.when(s == 15)
        def _rem():
            pltpu.sync_copy(table_h.at[pl.ds(9984, 16)], tab.at[pl.ds(9984, 16)])

        plsc.subcore_barrier()

        for t in range(n_idx):
            pltpu.sync_copy(idx_h[t].at[wid], idx_v)
            for b in range(NB):
                pltpu.async_copy(tab.at[idx_v.at[b]], bufs[b], sin[b])

            def wave(wv, carry, t=t):
                for b in range(NB):
                    j = wv * NB + b
                    pltpu.make_async_copy(tab.at[idx_v.at[j]], bufs[b], sin[b]).wait()
                    pltpu.async_copy(bufs[b], out_h[t].at[pl.ds((base + j) * CH, CH)], sout[b])
                for b in range(NB):
                    j = wv * NB + b
                    pltpu.make_async_copy(bufs[b], out_h[t].at[pl.ds((base + j) * CH, CH)], sout[b]).wait()

                    @pl.when(wv + 1 < NWAVE)
                    def _next(b=b, wv=wv):
                        jn = (wv + 1) * NB + b
                        pltpu.async_copy(tab.at[idx_v.at[jn]], bufs[b], sin[b])

                return carry

            lax.fori_loop(0, NWAVE, wave, 0)

    return gk


@functools.lru_cache(maxsize=None)
def _scatter_kernel():
    NBS = 2
    NWS = KCH // NBS
    mesh = plsc.VectorSubcoreMesh(core_axis_name="c", subcore_axis_name="s")

    @functools.partial(
        pl.kernel,
        out_type=jax.ShapeDtypeStruct((2, N, 128), F32),
        mesh=mesh,
        scratch_types=[
            pltpu.VMEM((KCH, CH), jnp.int32),
            *[pltpu.VMEM((CH, 128), F32) for _ in range(NBS)],
            *[pltpu.SemaphoreType.DMA for _ in range(NBS)],
            pltpu.VMEM_SHARED((NACC, 128), F32),
        ],
    )
    def sk(m_h, col_h, z_h, out_h, *scr):
        idx_v = scr[0]
        bufs = scr[1:1 + NBS]
        sin = scr[1 + NBS:1 + 2 * NBS]
        acc = scr[1 + 2 * NBS]
        c = lax.axis_index("c")
        s = lax.axis_index("s")
        wid = s * 2 + c
        base = wid * KCH
        pltpu.sync_copy(z_h, acc.at[pl.ds(s * ZR, ZR)])
        plsc.subcore_barrier()
        pltpu.sync_copy(col_h.at[wid], idx_v)
        for b in range(NBS):
            pltpu.async_copy(m_h.at[pl.ds((base + b) * CH, CH)], bufs[b], sin[b])

        def wave(wv, carry):
            for b in range(NBS):
                j = wv * NBS + b
                pltpu.make_async_copy(m_h.at[pl.ds((base + j) * CH, CH)], bufs[b], sin[b]).wait()
                pltpu.sync_copy(bufs[b], acc.at[idx_v.at[j]], add=True)

                @pl.when(wv + 1 < NWS)
                def _next(b=b, wv=wv):
                    jn = (wv + 1) * NBS + b
                    pltpu.async_copy(m_h.at[pl.ds((base + jn) * CH, CH)], bufs[b], sin[b])

            return carry

        lax.fori_loop(0, NWS, wave, 0)
        plsc.subcore_barrier()
        pltpu.sync_copy(acc.at[pl.ds(s * 624, 624)], out_h.at[c, pl.ds(s * 624, 624)])

        @pl.when(s == 15)
        def _rem():
            pltpu.sync_copy(acc.at[pl.ds(9984, 16)], out_h.at[c, pl.ds(9984, 16)])

    return sk


def _sc_gather(table, idx3s, D, dt=F32):
    out = _gather_kernel(len(idx3s), D, dt)(table, *idx3s)
    return out if isinstance(out, (tuple, list)) else (out,)


def _sc_scatter(m, col3, zeros):
    return _scatter_kernel()(m, col3, zeros)


def kernel(x, edge_index, edge_attr, params):
    x0 = x
    row = edge_index[0]
    col = edge_index[1]
    pad = EP - E
    row3 = jnp.concatenate([row, jnp.zeros((pad,), jnp.int32)]).reshape(NW, KCH, CH)
    col3 = jnp.concatenate([col, jnp.full((pad,), N, jnp.int32)]).reshape(NW, KCH, CH)
    eap = jnp.concatenate([edge_attr, jnp.zeros((pad, 16), F32)], axis=0)
    zeros = jnp.zeros((ZR, 128), F32)

    P = params
    w_ne = _mlp_weights(P["node_encoder"], "ne", [("a", 0, 128)])
    w_ee = _mlp_weights(P["edge_encoder"], "ee",
                        [("a", 0, 128), ("b", 128, 256), ("c", 256, 272)])
    w_ic = _mlp_weights(P["init_cls"], "ic",
                        [("a", 0, 64), ("b", 64, 128), ("c", 128, 192),
                         ("d", 192, 320), ("e", 320, 448), ("f", 448, 464)])
    w_gx = _mlp_weights(P["gat_x_mlp"], "gx",
                        [("a", 0, 64), ("b", 64, 192), ("c", 192, 320),
                         ("d", 320, 448)])
    w_gw = _mlp_weights(P["gat_w_mlp"], "gw", [("r", 0, 1), ("a", 1, 65)])
    w_em = _mlp_weights(P["edge_mlp"], "em",
                        [("a", 0, 64), ("b", 64, 128), ("c", 128, 192),
                         ("r", 192, 193), ("d", 193, 321), ("e", 321, 449),
                         ("f", 449, 465)])
    w_ec = _mlp_weights(P["edge_cls"], "ec",
                        [("a", 0, 64), ("b", 64, 128), ("c", 128, 192),
                         ("r", 192, 193), ("d", 193, 321), ("e", 321, 449),
                         ("f", 449, 465)])

    ixr, ixc = _sc_gather(x0, (row3, col3), 128)
    xe = _tc_call(_node_enc_body, GN, [x0], [_nspec(128)], w_ne,
                  [jax.ShapeDtypeStruct((N, 128), F32)], [_nspec(128)])[0]
    xr, xc = _sc_gather(xe, (row3, col3), 128)

    w_init = {}
    for dd in (w_ee, w_ic, w_gw):
        w_init.update(dd)
    for k in ("em1d", "em1e", "em1f", "emb1"):
        w_init[k] = w_em[k]
    for k in ("ec1d", "ec1e", "ec1f", "ecb1"):
        w_init[k] = w_ec[k]
    eh, el, c1, c2, w, m1 = _tc_call(
        _init_edge_body, GE,
        [ixr, ixc, eap, xr, xc],
        [_espec(128), _espec(128), _espec(16), _espec(128), _espec(128)],
        w_init,
        [jax.ShapeDtypeStruct((EP, 64), F32),
         jax.ShapeDtypeStruct((EP, 1), F32),
         jax.ShapeDtypeStruct((EP, 128), BF16),
         jax.ShapeDtypeStruct((EP, 128), BF16),
         jax.ShapeDtypeStruct((EP, 1), F32),
         jax.ShapeDtypeStruct((EP, 128), F32)],
        [_espec(64), _espec(1), _espec(128), _espec(128), _espec(1), _espec(128)])

    w_fused = {}
    for dd in (w_em, w_ec, w_gw):
        w_fused.update(dd)

    p2spec = pl.BlockSpec((2, TN, 128), lambda i: (0, i, 0))

    for _ in range(6):
        parts = _sc_scatter(m1, col3, zeros)
        one_hop = _tc_call(_sum_body, GN, [parts], [p2spec], {},
                           [jax.ShapeDtypeStruct((N, 128), F32)], [_nspec(128)])[0]
        (ohr,) = _sc_gather(one_hop, (row3,), 128)
        m2 = _tc_call(_m2_body, GE, [ohr, xc, eh, w],
                      [_espec(128), _espec(128), _espec(64), _espec(1)], {},
                      [jax.ShapeDtypeStruct((EP, 128), F32)], [_espec(128)])[0]
        parts2 = _sc_scatter(m2, col3, zeros)
        xe = _tc_call(_gat_x_body, GN, [parts2, one_hop, xe, x0],
                      [p2spec, _nspec(128), _nspec(128), _nspec(128)], w_gx,
                      [jax.ShapeDtypeStruct((N, 128), F32)], [_nspec(128)])[0]
        xr, xc = _sc_gather(xe, (row3, col3), 128)
        eh, el, w, m1 = _tc_call(
            _edge_fused_body, GE,
            [xr, xc, eh, el, c1, c2],
            [_espec(128), _espec(128), _espec(64), _espec(1), _espec(128), _espec(128)],
            w_fused,
            [jax.ShapeDtypeStruct((EP, 64), F32),
             jax.ShapeDtypeStruct((EP, 1), F32),
             jax.ShapeDtypeStruct((EP, 1), F32),
             jax.ShapeDtypeStruct((EP, 128), F32)],
            [_espec(64), _espec(1), _espec(1), _espec(128)])

    return el[:E, 0]
